# baseline (speedup 1.0000x reference)
import jax
import jax.numpy as jnp
from jax.experimental import pallas as pl
from jax.experimental.pallas import tpu as pltpu


def _pool2x2(r):
    H, W, B = r.shape
    rr = r.reshape(H // 2, 2, W, B)
    a = jnp.maximum(rr[:, 0], rr[:, 1])
    aa = a.reshape(H // 2, W // 2, 2, B)
    return jnp.maximum(aa[:, :, 0, :], aa[:, :, 1, :])


def _fernet_kernel(x_ref, t1_ref, b1_ref, w2_ref, b2_ref, w3_ref, b3_ref,
                   f1w_ref, f1b_ref, f2w_ref, f2b_ref, f3w_ref, f3b_ref,
                   o_ref, ys_ref, s2_ref, s3_ref, a1_ref, a2_ref, a3_ref):
    B = x_ref.shape[-1]

    t1v = t1_ref[...]

    def c1_rows(h, carry):
        ys_ref[h] = jnp.dot(t1v, x_ref[h],
                            preferred_element_type=jnp.float32)
        return carry

    jax.lax.fori_loop(0, 48, c1_rows, 0)

    def c1_comb(o, carry):
        r = [[None] * 22 for _ in range(2)]
        for kh in range(5):
            base = o * 240 + kh * 48
            for d in range(2):
                for hp in range(22):
                    t = ys_ref[2 * hp + d + kh, pl.ds(base, 48), :]
                    r[d][hp] = t if r[d][hp] is None else r[d][hp] + t
        b = b1_ref[o, 0]
        for hp in range(22):
            m = jnp.maximum(r[0][hp], r[1][hp])
            m = m.reshape(24, 2, B)
            m = jnp.maximum(m[:, 0, :], m[:, 1, :])
            a1_ref[o, hp] = jnp.maximum(m[:22] + b, 0.0)
        return carry

    jax.lax.fori_loop(0, 6, c1_comb, 0)

    for ci in range(6):
        plane = a1_ref[ci]
        for kh in range(3):
            for kw in range(3):
                f = (kh * 3 + kw) * 6 + ci
                s2_ref[f] = plane[kh:kh + 20, kw:kw + 20, :]

    def c2_body(co, carry):
        acc = w2_ref[co, 0] * s2_ref[0]
        for f in range(1, 54):
            acc = acc + w2_ref[co, f] * s2_ref[f]
        p = _pool2x2(jnp.maximum(acc + b2_ref[co, 0], 0.0))
        a2_ref[pl.ds(co, 1)] = p[None]
        return carry

    jax.lax.fori_loop(0, 6, c2_body, 0)

    for ci in range(6):
        plane = a2_ref[ci]
        for kh in range(3):
            for kw in range(3):
                f = (kh * 3 + kw) * 6 + ci
                s3_ref[f] = plane[kh:kh + 8, kw:kw + 8, :]

    def c3_body(co, carry):
        acc = w3_ref[co, 0] * s3_ref[0]
        for f in range(1, 54):
            acc = acc + w3_ref[co, f] * s3_ref[f]
        p = _pool2x2(jnp.maximum(acc + b3_ref[co, 0], 0.0))
        a3_ref[pl.ds(co, 1)] = p[None]
        return carry

    jax.lax.fori_loop(0, 16, c3_body, 0)

    xf = a3_ref[...].reshape(256, B)
    h = jax.lax.dot_general(f1w_ref[...], xf, (((0,), (0,)), ((), ())),
                            preferred_element_type=jnp.float32)
    h = jnp.maximum(h + f1b_ref[...], 0.0)
    h = jax.lax.dot_general(f2w_ref[...], h, (((0,), (0,)), ((), ())),
                            preferred_element_type=jnp.float32)
    h = jnp.maximum(h + f2b_ref[...], 0.0)
    o = jax.lax.dot_general(f3w_ref[...], h, (((0,), (0,)), ((), ())),
                            preferred_element_type=jnp.float32)
    o_ref[...] = (o + f3b_ref[...]).astype(o_ref.dtype)


def _fernet_call(xt, t1, c1b, c2w, c2b, c3w, c3b,
                 f1w, f1bc, f2w, f2bc, f3w, f3bc, *, interpret=False):
    N = xt.shape[-1]
    B = 128

    def smem(arr):
        return pl.BlockSpec(memory_space=pltpu.SMEM)

    def resident(arr):
        return pl.BlockSpec(arr.shape, lambda j: (0,) * arr.ndim)

    return pl.pallas_call(
        _fernet_kernel,
        out_shape=jax.ShapeDtypeStruct((3, N), jnp.float32),
        grid=(N // B,),
        in_specs=[pl.BlockSpec((48, 48, B), lambda j: (0, 0, j)),
                  resident(t1), smem(c1b),
                  smem(c2w), smem(c2b),
                  smem(c3w), smem(c3b),
                  resident(f1w), resident(f1bc),
                  resident(f2w), resident(f2bc),
                  resident(f3w), resident(f3bc)],
        out_specs=pl.BlockSpec((3, B), lambda j: (0, j)),
        scratch_shapes=[pltpu.VMEM((48, 1440, B), jnp.float32),
                        pltpu.VMEM((54, 20, 20, B), jnp.float32),
                        pltpu.VMEM((54, 8, 8, B), jnp.float32),
                        pltpu.VMEM((6, 22, 22, B), jnp.float32),
                        pltpu.VMEM((6, 10, 10, B), jnp.float32),
                        pltpu.VMEM((16, 4, 4, B), jnp.float32)],
        compiler_params=pltpu.CompilerParams(
            dimension_semantics=("arbitrary",)),
        interpret=interpret,
    )(xt, t1, c1b, c2w, c2b, c3w, c3b, f1w, f1bc, f2w, f2bc, f3w, f3bc)


def _toeplitz_w1(c1w):
    w1r = c1w.reshape(6, 5, 5)
    eyes = jnp.stack([jnp.eye(48, 48, k, dtype=jnp.float32)
                      for k in range(5)])
    t1 = jnp.einsum('oht,tab->ohab', w1r, eyes)
    return t1.reshape(1440, 48).astype(jnp.bfloat16)


def kernel(x, c1w, c1b, c2w, c2b, c3w, c3b, f1w, f1b, f2w, f2b, f3w, f3b):
    N = x.shape[0]
    xt = jnp.transpose(x.reshape(N, 48, 48), (1, 2, 0)).astype(jnp.bfloat16)
    out = _fernet_call(xt, _toeplitz_w1(c1w), c1b, c2w, c2b, c3w, c3b,
                       f1w, f1b.T, f2w, f2b.T, f3w, f3b.T)
    return out.T


# speedup vs baseline: 1.7451x; 1.7451x over previous
import jax
import jax.numpy as jnp
from jax.experimental import pallas as pl
from jax.experimental.pallas import tpu as pltpu


def _pool2x2(r):
    H, W, B = r.shape
    rr = r.reshape(H // 2, 2, W, B)
    a = jnp.maximum(rr[:, 0], rr[:, 1])
    aa = a.reshape(H // 2, W // 2, 2, B)
    return jnp.maximum(aa[:, :, 0, :], aa[:, :, 1, :])


def _fernet_kernel(x_ref, t1_ref, b1_ref, w2_ref, b2_ref, w3_ref, b3_ref,
                   f1w_ref, f1b_ref, f2w_ref, f2b_ref, f3w_ref, f3b_ref,
                   o_ref, s2_ref, s3_ref, a1_ref, a2_ref, a3_ref):
    B = x_ref.shape[-1]

    xv = x_ref[...]
    t1s = [t1_ref[kh] for kh in range(5)]
    b1v = b1_ref[...].reshape(6, 1, B)

    for hp in range(22):
        y0 = jnp.dot(t1s[0], xv[2 * hp],
                     preferred_element_type=jnp.float32)
        y1 = jnp.dot(t1s[0], xv[2 * hp + 1],
                     preferred_element_type=jnp.float32)
        for kh in range(1, 5):
            y0 = y0 + jnp.dot(t1s[kh], xv[2 * hp + kh],
                              preferred_element_type=jnp.float32)
            y1 = y1 + jnp.dot(t1s[kh], xv[2 * hp + 1 + kh],
                              preferred_element_type=jnp.float32)
        m = jnp.maximum(y0, y1)
        m = m.reshape(6, 24, 2, B)
        m = jnp.maximum(m[:, :, 0, :], m[:, :, 1, :])
        a1_ref[:, hp] = jnp.maximum(m[:, :22, :] + b1v, 0.0)

    for ci in range(6):
        plane = a1_ref[ci]
        for kh in range(3):
            for kw in range(3):
                f = (kh * 3 + kw) * 6 + ci
                s2_ref[f] = plane[kh:kh + 20, kw:kw + 20, :]

    def c2_body(co, carry):
        acc = w2_ref[co, 0] * s2_ref[0]
        for f in range(1, 54):
            acc = acc + w2_ref[co, f] * s2_ref[f]
        p = _pool2x2(jnp.maximum(acc + b2_ref[co, 0], 0.0))
        a2_ref[pl.ds(co, 1)] = p[None]
        return carry

    jax.lax.fori_loop(0, 6, c2_body, 0)

    for ci in range(6):
        plane = a2_ref[ci]
        for kh in range(3):
            for kw in range(3):
                f = (kh * 3 + kw) * 6 + ci
                s3_ref[f] = plane[kh:kh + 8, kw:kw + 8, :]

    def c3_body(co, carry):
        acc = w3_ref[co, 0] * s3_ref[0]
        for f in range(1, 54):
            acc = acc + w3_ref[co, f] * s3_ref[f]
        p = _pool2x2(jnp.maximum(acc + b3_ref[co, 0], 0.0))
        a3_ref[pl.ds(co, 1)] = p[None]
        return carry

    jax.lax.fori_loop(0, 16, c3_body, 0)

    xf = a3_ref[...].reshape(256, B)
    h = jax.lax.dot_general(f1w_ref[...], xf, (((0,), (0,)), ((), ())),
                            preferred_element_type=jnp.float32)
    h = jnp.maximum(h + f1b_ref[...], 0.0)
    h = jax.lax.dot_general(f2w_ref[...], h, (((0,), (0,)), ((), ())),
                            preferred_element_type=jnp.float32)
    h = jnp.maximum(h + f2b_ref[...], 0.0)
    o = jax.lax.dot_general(f3w_ref[...], h, (((0,), (0,)), ((), ())),
                            preferred_element_type=jnp.float32)
    o_ref[...] = (o + f3b_ref[...]).astype(o_ref.dtype)


def _fernet_call(xt, t1, c1b, c2w, c2b, c3w, c3b,
                 f1w, f1bc, f2w, f2bc, f3w, f3bc, *, interpret=False):
    N = xt.shape[-1]
    B = 128

    def smem(arr):
        return pl.BlockSpec(memory_space=pltpu.SMEM)

    def resident(arr):
        return pl.BlockSpec(arr.shape, lambda j: (0,) * arr.ndim)

    return pl.pallas_call(
        _fernet_kernel,
        out_shape=jax.ShapeDtypeStruct((3, N), jnp.float32),
        grid=(N // B,),
        in_specs=[pl.BlockSpec((48, 48, B), lambda j: (0, 0, j)),
                  resident(t1), resident(c1b),
                  smem(c2w), smem(c2b),
                  smem(c3w), smem(c3b),
                  resident(f1w), resident(f1bc),
                  resident(f2w), resident(f2bc),
                  resident(f3w), resident(f3bc)],
        out_specs=pl.BlockSpec((3, B), lambda j: (0, j)),
        scratch_shapes=[pltpu.VMEM((54, 20, 20, B), jnp.float32),
                        pltpu.VMEM((54, 8, 8, B), jnp.float32),
                        pltpu.VMEM((6, 22, 22, B), jnp.float32),
                        pltpu.VMEM((6, 10, 10, B), jnp.float32),
                        pltpu.VMEM((16, 4, 4, B), jnp.float32)],
        compiler_params=pltpu.CompilerParams(
            dimension_semantics=("arbitrary",)),
        interpret=interpret,
    )(xt, t1, c1b, c2w, c2b, c3w, c3b, f1w, f1bc, f2w, f2bc, f3w, f3bc)


def _toeplitz_w1(c1w):
    w1r = c1w.reshape(6, 5, 5)
    eyes = jnp.stack([jnp.eye(48, 48, k, dtype=jnp.float32)
                      for k in range(5)])
    t1 = jnp.einsum('oht,tab->hoab', w1r, eyes)
    return t1.reshape(5, 288, 48).astype(jnp.bfloat16)


def kernel(x, c1w, c1b, c2w, c2b, c3w, c3b, f1w, f1b, f2w, f2b, f3w, f3b):
    N = x.shape[0]
    xt = jnp.transpose(x.reshape(N, 48, 48), (1, 2, 0)).astype(jnp.bfloat16)
    c1b_lanes = jnp.tile(c1b, (1, 128))
    out = _fernet_call(xt, _toeplitz_w1(c1w), c1b_lanes, c2w, c2b, c3w, c3b,
                       f1w, f1b.T, f2w, f2b.T, f3w, f3b.T)
    return out.T


# speedup vs baseline: 3.0561x; 1.7512x over previous
import jax
import jax.numpy as jnp
from jax.experimental import pallas as pl
from jax.experimental.pallas import tpu as pltpu


def _fernet_kernel(x_ref, t1_ref, b1_ref, t2_ref, b2_ref, t3_ref, b3_ref,
                   f1w_ref, f1b_ref, f2w_ref, f2b_ref, f3w_ref, f3b_ref,
                   o_ref, a1_ref, a2_ref, a3_ref):
    B = x_ref.shape[-1]
    f32 = jnp.float32

    xv = x_ref[...]
    t1s = [t1_ref[kh] for kh in range(5)]
    b1v = b1_ref[...].reshape(6, 1, B)

    for hp in range(22):
        y0 = jnp.dot(t1s[0], xv[2 * hp], preferred_element_type=f32)
        y1 = jnp.dot(t1s[0], xv[2 * hp + 1], preferred_element_type=f32)
        for kh in range(1, 5):
            y0 = y0 + jnp.dot(t1s[kh], xv[2 * hp + kh],
                              preferred_element_type=f32)
            y1 = y1 + jnp.dot(t1s[kh], xv[2 * hp + 1 + kh],
                              preferred_element_type=f32)
        m = jnp.maximum(y0, y1)
        m = m.reshape(6, 24, 2, B)
        m = jnp.maximum(m[:, :, 0, :], m[:, :, 1, :])
        m = jnp.maximum(m[:, :22, :] + b1v, 0.0)
        a1_ref[:, hp] = m.astype(jnp.bfloat16)

    t2s = [t2_ref[kh] for kh in range(3)]
    b2v = b2_ref[...].reshape(6, 1, B)

    def x2row(h):
        return a1_ref[:, h].reshape(132, B)

    for hp in range(10):
        y0 = jnp.dot(t2s[0], x2row(2 * hp), preferred_element_type=f32)
        y1 = jnp.dot(t2s[0], x2row(2 * hp + 1), preferred_element_type=f32)
        for kh in range(1, 3):
            y0 = y0 + jnp.dot(t2s[kh], x2row(2 * hp + kh),
                              preferred_element_type=f32)
            y1 = y1 + jnp.dot(t2s[kh], x2row(2 * hp + 1 + kh),
                              preferred_element_type=f32)
        m = jnp.maximum(y0, y1)
        m = m.reshape(6, 12, 2, B)
        m = jnp.maximum(m[:, :, 0, :], m[:, :, 1, :])
        m = jnp.maximum(m[:, :10, :] + b2v, 0.0)
        a2_ref[:, hp] = m.astype(jnp.bfloat16)

    t3s = [t3_ref[kh] for kh in range(3)]
    b3v = b3_ref[...].reshape(16, 1, B)

    def x3row(h):
        return a2_ref[:, h].reshape(60, B)

    for hp in range(4):
        y0 = jnp.dot(t3s[0], x3row(2 * hp), preferred_element_type=f32)
        y1 = jnp.dot(t3s[0], x3row(2 * hp + 1), preferred_element_type=f32)
        for kh in range(1, 3):
            y0 = y0 + jnp.dot(t3s[kh], x3row(2 * hp + kh),
                              preferred_element_type=f32)
            y1 = y1 + jnp.dot(t3s[kh], x3row(2 * hp + 1 + kh),
                              preferred_element_type=f32)
        m = jnp.maximum(y0, y1)
        m = m.reshape(16, 4, 2, B)
        m = jnp.maximum(m[:, :, 0, :], m[:, :, 1, :])
        a3_ref[:, hp] = jnp.maximum(m + b3v, 0.0)

    xf = a3_ref[...].reshape(256, B)
    h = jax.lax.dot_general(f1w_ref[...], xf, (((0,), (0,)), ((), ())),
                            preferred_element_type=f32)
    h = jnp.maximum(h + f1b_ref[...], 0.0)
    h = jax.lax.dot_general(f2w_ref[...], h, (((0,), (0,)), ((), ())),
                            preferred_element_type=f32)
    h = jnp.maximum(h + f2b_ref[...], 0.0)
    o = jax.lax.dot_general(f3w_ref[...], h, (((0,), (0,)), ((), ())),
                            preferred_element_type=f32)
    o_ref[...] = (o + f3b_ref[...]).astype(o_ref.dtype)


def _fernet_call(xt, t1, b1l, t2, b2l, t3, b3l,
                 f1w, f1bc, f2w, f2bc, f3w, f3bc, *, interpret=False):
    N = xt.shape[-1]
    B = 128

    def resident(arr):
        return pl.BlockSpec(arr.shape, lambda j: (0,) * arr.ndim)

    return pl.pallas_call(
        _fernet_kernel,
        out_shape=jax.ShapeDtypeStruct((3, N), jnp.float32),
        grid=(N // B,),
        in_specs=[pl.BlockSpec((48, 48, B), lambda j: (0, 0, j)),
                  resident(t1), resident(b1l),
                  resident(t2), resident(b2l),
                  resident(t3), resident(b3l),
                  resident(f1w), resident(f1bc),
                  resident(f2w), resident(f2bc),
                  resident(f3w), resident(f3bc)],
        out_specs=pl.BlockSpec((3, B), lambda j: (0, j)),
        scratch_shapes=[pltpu.VMEM((6, 22, 22, B), jnp.bfloat16),
                        pltpu.VMEM((6, 10, 10, B), jnp.bfloat16),
                        pltpu.VMEM((16, 4, 4, B), jnp.float32)],
        compiler_params=pltpu.CompilerParams(
            dimension_semantics=("arbitrary",)),
        interpret=interpret,
    )(xt, t1, b1l, t2, b2l, t3, b3l, f1w, f1bc, f2w, f2bc, f3w, f3bc)


def _toeplitz(w4, wo_pad, w_in):
    cout, K, _, cin = w4.shape
    eyes = jnp.stack([jnp.eye(wo_pad, w_in, k, dtype=jnp.float32)
                      for k in range(K)])
    t = jnp.einsum('oktc,tab->koacb', w4, eyes)
    return t.reshape(K, cout * wo_pad, cin * w_in).astype(jnp.bfloat16)


def kernel(x, c1w, c1b, c2w, c2b, c3w, c3b, f1w, f1b, f2w, f2b, f3w, f3b):
    N = x.shape[0]
    xt = jnp.transpose(x.reshape(N, 48, 48), (1, 2, 0)).astype(jnp.bfloat16)
    t1 = _toeplitz(c1w.reshape(6, 5, 5, 1), 48, 48)
    t2 = _toeplitz(c2w.reshape(6, 3, 3, 6), 24, 22)
    t3 = _toeplitz(c3w.reshape(16, 3, 3, 6), 8, 10)
    lanes = lambda b: jnp.tile(b, (1, 128))
    out = _fernet_call(xt, t1, lanes(c1b), t2, lanes(c2b), t3, lanes(c3b),
                       f1w, f1b.T, f2w, f2b.T, f3w, f3b.T)
    return out.T


# speedup vs baseline: 3.8347x; 1.2548x over previous
import jax
import jax.numpy as jnp
from jax.experimental import pallas as pl
from jax.experimental.pallas import tpu as pltpu


def _fernet_kernel(x_ref, t1_ref, b1_ref, t2_ref, b2_ref, t3_ref, b3_ref,
                   f1w_ref, f1b_ref, f2w_ref, f2b_ref, f3w_ref, f3b_ref,
                   o_ref, a1_ref, a2_ref, a3_ref):
    B = x_ref.shape[-1]
    f32 = jnp.float32

    xv = x_ref[...]
    t1s = [t1_ref[kh] for kh in range(5)]
    b1v = b1_ref[...].reshape(6, 1, B)

    for hp in range(22):
        y0 = jnp.dot(t1s[0], xv[2 * hp], preferred_element_type=f32)
        y1 = jnp.dot(t1s[0], xv[2 * hp + 1], preferred_element_type=f32)
        for kh in range(1, 5):
            y0 = y0 + jnp.dot(t1s[kh], xv[2 * hp + kh],
                              preferred_element_type=f32)
            y1 = y1 + jnp.dot(t1s[kh], xv[2 * hp + 1 + kh],
                              preferred_element_type=f32)
        m = jnp.maximum(y0, y1)
        m = m.reshape(6, 24, 2, B)
        m = jnp.maximum(m[:, :, 0, :], m[:, :, 1, :])
        m = jnp.maximum(m[:, :22, :] + b1v, 0.0)
        a1_ref[:, hp] = m.astype(jnp.bfloat16)

    t2s = [t2_ref[kh] for kh in range(3)]
    b2v = b2_ref[...].reshape(6, 1, B)

    def x2row(h):
        return a1_ref[:, h].reshape(132, B)

    for hp in range(10):
        y0 = jnp.dot(t2s[0], x2row(2 * hp), preferred_element_type=f32)
        y1 = jnp.dot(t2s[0], x2row(2 * hp + 1), preferred_element_type=f32)
        for kh in range(1, 3):
            y0 = y0 + jnp.dot(t2s[kh], x2row(2 * hp + kh),
                              preferred_element_type=f32)
            y1 = y1 + jnp.dot(t2s[kh], x2row(2 * hp + 1 + kh),
                              preferred_element_type=f32)
        m = jnp.maximum(y0, y1)
        m = m.reshape(6, 12, 2, B)
        m = jnp.maximum(m[:, :, 0, :], m[:, :, 1, :])
        m = jnp.maximum(m[:, :10, :] + b2v, 0.0)
        a2_ref[:, hp] = m.astype(jnp.bfloat16)

    t3s = [t3_ref[kh] for kh in range(3)]
    b3v = b3_ref[...].reshape(16, 1, B)

    def x3row(h):
        return a2_ref[:, h].reshape(60, B)

    for hp in range(4):
        y0 = jnp.dot(t3s[0], x3row(2 * hp), preferred_element_type=f32)
        y1 = jnp.dot(t3s[0], x3row(2 * hp + 1), preferred_element_type=f32)
        for kh in range(1, 3):
            y0 = y0 + jnp.dot(t3s[kh], x3row(2 * hp + kh),
                              preferred_element_type=f32)
            y1 = y1 + jnp.dot(t3s[kh], x3row(2 * hp + 1 + kh),
                              preferred_element_type=f32)
        m = jnp.maximum(y0, y1)
        m = m.reshape(16, 4, 2, B)
        m = jnp.maximum(m[:, :, 0, :], m[:, :, 1, :])
        a3_ref[:, hp] = jnp.maximum(m + b3v, 0.0)

    xf = a3_ref[...].reshape(256, B)
    h = jax.lax.dot_general(f1w_ref[...], xf, (((0,), (0,)), ((), ())),
                            preferred_element_type=f32)
    h = jnp.maximum(h + f1b_ref[...], 0.0)
    h = jax.lax.dot_general(f2w_ref[...], h, (((0,), (0,)), ((), ())),
                            preferred_element_type=f32)
    h = jnp.maximum(h + f2b_ref[...], 0.0)
    o = jax.lax.dot_general(f3w_ref[...], h, (((0,), (0,)), ((), ())),
                            preferred_element_type=f32)
    o_ref[...] = (o + f3b_ref[...]).astype(o_ref.dtype)


def _fernet_call(xt, t1, b1l, t2, b2l, t3, b3l,
                 f1w, f1bc, f2w, f2bc, f3w, f3bc, *, interpret=False):
    N = xt.shape[-1]
    B = 256

    def resident(arr):
        return pl.BlockSpec(arr.shape, lambda j: (0,) * arr.ndim)

    return pl.pallas_call(
        _fernet_kernel,
        out_shape=jax.ShapeDtypeStruct((3, N), jnp.float32),
        grid=(N // B,),
        in_specs=[pl.BlockSpec((48, 48, B), lambda j: (0, 0, j)),
                  resident(t1), resident(b1l),
                  resident(t2), resident(b2l),
                  resident(t3), resident(b3l),
                  resident(f1w), resident(f1bc),
                  resident(f2w), resident(f2bc),
                  resident(f3w), resident(f3bc)],
        out_specs=pl.BlockSpec((3, B), lambda j: (0, j)),
        scratch_shapes=[pltpu.VMEM((6, 22, 22, B), jnp.bfloat16),
                        pltpu.VMEM((6, 10, 10, B), jnp.bfloat16),
                        pltpu.VMEM((16, 4, 4, B), jnp.float32)],
        compiler_params=pltpu.CompilerParams(
            dimension_semantics=("arbitrary",)),
        interpret=interpret,
    )(xt, t1, b1l, t2, b2l, t3, b3l, f1w, f1bc, f2w, f2bc, f3w, f3bc)


def _toeplitz(w4, wo_pad, w_in):
    cout, K, _, cin = w4.shape
    eyes = jnp.stack([jnp.eye(wo_pad, w_in, k, dtype=jnp.float32)
                      for k in range(K)])
    t = jnp.einsum('oktc,tab->koacb', w4, eyes)
    return t.reshape(K, cout * wo_pad, cin * w_in).astype(jnp.bfloat16)


def kernel(x, c1w, c1b, c2w, c2b, c3w, c3b, f1w, f1b, f2w, f2b, f3w, f3b):
    N = x.shape[0]
    xt = jnp.transpose(x.reshape(N, 48, 48), (1, 2, 0)).astype(jnp.bfloat16)
    t1 = _toeplitz(c1w.reshape(6, 5, 5, 1), 48, 48)
    t2 = _toeplitz(c2w.reshape(6, 3, 3, 6), 24, 22)
    t3 = _toeplitz(c3w.reshape(16, 3, 3, 6), 8, 10)
    lanes = lambda b: jnp.tile(b, (1, 256))
    out = _fernet_call(xt, t1, lanes(c1b), t2, lanes(c2b), t3, lanes(c3b),
                       f1w, f1b.T, f2w, f2b.T, f3w, f3b.T)
    return out.T


# speedup vs baseline: 4.0513x; 1.0565x over previous
import jax
import jax.numpy as jnp
from jax.experimental import pallas as pl
from jax.experimental.pallas import tpu as pltpu


def _fernet_kernel(x_ref, t1_ref, b1_ref, t2_ref, b2_ref, t3_ref, b3_ref,
                   f1w_ref, f1b_ref, f2w_ref, f2b_ref, f3w_ref, f3b_ref,
                   o_ref, a1_ref, a2_ref, a3_ref):
    B = x_ref.shape[-1]
    f32 = jnp.float32

    xv = x_ref[...]
    t1s = [t1_ref[kh] for kh in range(5)]
    b1v = b1_ref[...].reshape(6, 1, B)

    for hp in range(22):
        y0 = jnp.dot(t1s[0], xv[2 * hp], preferred_element_type=f32)
        y1 = jnp.dot(t1s[0], xv[2 * hp + 1], preferred_element_type=f32)
        for kh in range(1, 5):
            y0 = y0 + jnp.dot(t1s[kh], xv[2 * hp + kh],
                              preferred_element_type=f32)
            y1 = y1 + jnp.dot(t1s[kh], xv[2 * hp + 1 + kh],
                              preferred_element_type=f32)
        m = jnp.maximum(y0, y1)
        m = m.reshape(6, 24, 2, B)
        m = jnp.maximum(m[:, :, 0, :], m[:, :, 1, :])
        m = jnp.maximum(m[:, :22, :] + b1v, 0.0)
        a1_ref[:, hp] = m.astype(jnp.bfloat16)

    t2s = [t2_ref[kh] for kh in range(3)]
    b2v = b2_ref[...].reshape(6, 1, B)

    def x2row(h):
        return a1_ref[:, h].reshape(132, B)

    for hp in range(10):
        y0 = jnp.dot(t2s[0], x2row(2 * hp), preferred_element_type=f32)
        y1 = jnp.dot(t2s[0], x2row(2 * hp + 1), preferred_element_type=f32)
        for kh in range(1, 3):
            y0 = y0 + jnp.dot(t2s[kh], x2row(2 * hp + kh),
                              preferred_element_type=f32)
            y1 = y1 + jnp.dot(t2s[kh], x2row(2 * hp + 1 + kh),
                              preferred_element_type=f32)
        m = jnp.maximum(y0, y1)
        m = m.reshape(6, 12, 2, B)
        m = jnp.maximum(m[:, :, 0, :], m[:, :, 1, :])
        m = jnp.maximum(m[:, :10, :] + b2v, 0.0)
        a2_ref[:, hp] = m.astype(jnp.bfloat16)

    t3s = [t3_ref[kh] for kh in range(3)]
    b3v = b3_ref[...].reshape(16, 1, B)

    def x3row(h):
        return a2_ref[:, h].reshape(60, B)

    for hp in range(4):
        y0 = jnp.dot(t3s[0], x3row(2 * hp), preferred_element_type=f32)
        y1 = jnp.dot(t3s[0], x3row(2 * hp + 1), preferred_element_type=f32)
        for kh in range(1, 3):
            y0 = y0 + jnp.dot(t3s[kh], x3row(2 * hp + kh),
                              preferred_element_type=f32)
            y1 = y1 + jnp.dot(t3s[kh], x3row(2 * hp + 1 + kh),
                              preferred_element_type=f32)
        m = jnp.maximum(y0, y1)
        m = m.reshape(16, 4, 2, B)
        m = jnp.maximum(m[:, :, 0, :], m[:, :, 1, :])
        a3_ref[:, hp] = jnp.maximum(m + b3v, 0.0)

    xf = a3_ref[...].reshape(256, B)
    h = jax.lax.dot_general(f1w_ref[...], xf, (((0,), (0,)), ((), ())),
                            preferred_element_type=f32)
    h = jnp.maximum(h + f1b_ref[...], 0.0)
    h = jax.lax.dot_general(f2w_ref[...], h, (((0,), (0,)), ((), ())),
                            preferred_element_type=f32)
    h = jnp.maximum(h + f2b_ref[...], 0.0)
    o = jax.lax.dot_general(f3w_ref[...], h, (((0,), (0,)), ((), ())),
                            preferred_element_type=f32)
    o_ref[...] = (o + f3b_ref[...]).astype(o_ref.dtype)


def _fernet_call(xt, t1, b1l, t2, b2l, t3, b3l,
                 f1w, f1bc, f2w, f2bc, f3w, f3bc, *, interpret=False):
    N = xt.shape[-1]
    B = 512

    def resident(arr):
        return pl.BlockSpec(arr.shape, lambda j: (0,) * arr.ndim)

    return pl.pallas_call(
        _fernet_kernel,
        out_shape=jax.ShapeDtypeStruct((3, N), jnp.float32),
        grid=(N // B,),
        in_specs=[pl.BlockSpec((48, 48, B), lambda j: (0, 0, j)),
                  resident(t1), resident(b1l),
                  resident(t2), resident(b2l),
                  resident(t3), resident(b3l),
                  resident(f1w), resident(f1bc),
                  resident(f2w), resident(f2bc),
                  resident(f3w), resident(f3bc)],
        out_specs=pl.BlockSpec((3, B), lambda j: (0, j)),
        scratch_shapes=[pltpu.VMEM((6, 22, 22, B), jnp.bfloat16),
                        pltpu.VMEM((6, 10, 10, B), jnp.bfloat16),
                        pltpu.VMEM((16, 4, 4, B), jnp.float32)],
        compiler_params=pltpu.CompilerParams(
            dimension_semantics=("arbitrary",)),
        interpret=interpret,
    )(xt, t1, b1l, t2, b2l, t3, b3l, f1w, f1bc, f2w, f2bc, f3w, f3bc)


def _toeplitz(w4, wo_pad, w_in):
    cout, K, _, cin = w4.shape
    eyes = jnp.stack([jnp.eye(wo_pad, w_in, k, dtype=jnp.float32)
                      for k in range(K)])
    t = jnp.einsum('oktc,tab->koacb', w4, eyes)
    return t.reshape(K, cout * wo_pad, cin * w_in).astype(jnp.bfloat16)


def kernel(x, c1w, c1b, c2w, c2b, c3w, c3b, f1w, f1b, f2w, f2b, f3w, f3b):
    N = x.shape[0]
    xt = jnp.transpose(x.reshape(N, 48, 48), (1, 2, 0)).astype(jnp.bfloat16)
    t1 = _toeplitz(c1w.reshape(6, 5, 5, 1), 48, 48)
    t2 = _toeplitz(c2w.reshape(6, 3, 3, 6), 24, 22)
    t3 = _toeplitz(c3w.reshape(16, 3, 3, 6), 8, 10)
    lanes = lambda b: jnp.tile(b, (1, 512))
    out = _fernet_call(xt, t1, lanes(c1b), t2, lanes(c2b), t3, lanes(c3b),
                       f1w, f1b.T, f2w, f2b.T, f3w, f3b.T)
    return out.T
